# baseline (speedup 1.0000x reference)
import jax
import jax.numpy as jnp
from jax import lax
from jax.experimental import pallas as pl
from jax.experimental.pallas import tpu as pltpu
from jax.experimental.pallas import tpu_sc as plsc

_VOCAB = 1000000
_EMBED = 32
_SCALE = float(_EMBED) ** 0.5

_NC = 2
_NS = 16
_NW = _NC * _NS

_ROWS = 16384
_SEQ = 50
_EB = _EMBED // 8
_IT = _ROWS // 128
_ITW = _IT // _NW
_PW = 5
_NP = _SEQ // _PW
_NCHUNK = _ITW * _NP
_CROWS = 128 * _PW
_GROUPS = _CROWS // 16


def _rsqrt(x):
    i = plsc.bitcast(x, jnp.int32)
    i = jnp.int32(0x5F3759DF) - lax.shift_right_logical(i, 1)
    y = plsc.bitcast(i, jnp.float32)
    for _ in range(3):
        y = y * (1.5 - 0.5 * x * y * y)
    return y


def _sc_kernel_body(weight_hbm, idx_hbm, out_hbm,
                    idxr0, idxr1, idxt0, idxt1, rows0, rows1, st0, st1,
                    gsem0, gsem1, wsem0, wsem1):
    wid = lax.axis_index("s") * _NC + lax.axis_index("c")
    iota16 = lax.iota(jnp.int32, 16)
    zero16 = jnp.zeros((16,), jnp.int32)
    idxr_b = (idxr0, idxr1)
    idxt_b = (idxt0, idxt1)
    rows_b = (rows0, rows1)
    st_b = (st0, st1)
    gsem_b = (gsem0, gsem1)
    wsem_b = (wsem0, wsem1)

    def split(ci):
        ibl = ci // _NP
        p0 = (ci - ibl * _NP) * _PW
        return ibl, p0

    def fire_gathers(ci, b):
        ibl, p0 = split(ci)
        i0 = pl.multiple_of((wid * _ITW + ibl) * 128, 8)
        pltpu.sync_copy(idx_hbm.at[pl.ds(i0, 128)], idxr_b[b])
        for jj in range(_PW):
            pv = jnp.full((16,), p0 + jj, jnp.int32)
            for t8 in range(8):
                tv = t8 * 16 + iota16
                v = plsc.load_gather(idxr_b[b], [tv, pv])
                idxt_b[b][jj, pl.ds(t8 * 16, 16)] = v
        for j in range(_PW):
            pltpu.async_copy(
                weight_hbm.at[idxt_b[b].at[j]], rows_b[b].at[j], gsem_b[b]
            )

    def drain_gathers(b):
        for j in range(_PW):
            pltpu.make_async_copy(
                weight_hbm.at[idxt_b[b].at[j]], rows_b[b].at[j], gsem_b[b]
            ).wait()

    def wb_copy(ci, b):
        ibl, p0 = split(ci)
        ib = wid * _ITW + ibl
        return pltpu.make_async_copy(
            st_b[b],
            out_hbm.at[pl.ds(p0, _PW), pl.ds(0, _EB), pl.ds(ib, 1)],
            wsem_b[b],
        )

    def compute(b):
        rows_v = rows_b[b]
        st_v = st_b[b]

        def group_body(g, _):
            j = g // 8
            jv = jnp.full((16,), j, jnp.int32)
            tv = (g - j * 8) * 16 + iota16
            cols = []
            acc = jnp.full((16,), 1e-24, jnp.float32)
            diag = [(iota16 + k) & (_EMBED - 1) for k in range(_EMBED)]
            for k in range(_EMBED):
                v = plsc.load_gather(rows_v, [jv, tv, diag[k]])
                cols.append(v)
                acc = acc + v * v
            scale = _rsqrt(acc) * _SCALE
            for k in range(_EMBED):
                e = diag[k]
                plsc.store_scatter(
                    st_v,
                    [jv, lax.shift_right_logical(e, 3), zero16, e & 7, tv],
                    cols[k] * scale,
                )
            return 0

        lax.fori_loop(0, _GROUPS, group_body, 0)

    def half(ci, b):
        b2 = 1 - b

        @pl.when(ci + 1 < _NCHUNK)
        def _():
            fire_gathers(ci + 1, b2)

        drain_gathers(b)

        @pl.when(ci >= 2)
        def _():
            wb_copy(ci - 2, b).wait()

        compute(b)
        wb_copy(ci, b).start()

    fire_gathers(0, 0)

    def pair_body(k, _):
        half(2 * k, 0)
        half(2 * k + 1, 1)
        return 0

    lax.fori_loop(0, _NCHUNK // 2, pair_body, 0)
    wb_copy(_NCHUNK - 2, 0).wait()
    wb_copy(_NCHUNK - 1, 1).wait()


@jax.jit
def _run(weight, idx):
    mesh = plsc.VectorSubcoreMesh(core_axis_name="c", subcore_axis_name="s")
    f = pl.kernel(
        _sc_kernel_body,
        out_type=jax.ShapeDtypeStruct((_SEQ, _EB, _IT, 8, 128), jnp.float32),
        mesh=mesh,
        scratch_types=[
            pltpu.VMEM((128, _SEQ), jnp.int32),
            pltpu.VMEM((128, _SEQ), jnp.int32),
            pltpu.VMEM((_PW, 128), jnp.int32),
            pltpu.VMEM((_PW, 128), jnp.int32),
            pltpu.VMEM((_PW, 128, _EMBED), jnp.float32),
            pltpu.VMEM((_PW, 128, _EMBED), jnp.float32),
            pltpu.VMEM((_PW, _EB, 1, 8, 128), jnp.float32),
            pltpu.VMEM((_PW, _EB, 1, 8, 128), jnp.float32),
            pltpu.SemaphoreType.DMA,
            pltpu.SemaphoreType.DMA,
            pltpu.SemaphoreType.DMA,
            pltpu.SemaphoreType.DMA,
        ],
        compiler_params=pltpu.CompilerParams(
            needs_layout_passes=False, use_tc_tiling_on_sc=False
        ),
    )
    return f(weight, idx)


def kernel(x, weight):
    b = _run(weight, x.astype(jnp.int32))
    return b.transpose(2, 4, 0, 1, 3).reshape(_ROWS, _SEQ, _EMBED)
